# Initial kernel scaffold; baseline (speedup 1.0000x reference)
import functools
import numpy as np
import jax
import jax.numpy as jnp
from jax import lax
from jax.experimental import pallas as pl
from jax.experimental.pallas import tpu as pltpu
from jax.experimental.pallas import tpu_sc as plsc

N = 10000
E = 320000
NW = 32
EPW = E // NW
CH = 80
NCH = EPW // CH
NBLK = 2000
EBLK = 2000


def _sel_mats():
    P_r = np.zeros((3, 16, 32), np.float32)
    Q_r = np.zeros((3, 16, 32), np.float32)
    P_d = np.zeros((3, 16, 32), np.float32)
    Q_d = np.zeros((3, 16, 32), np.float32)
    for k in range(3):
        for i in range(4):
            for j in range(4):
                P_r[k, 3 * i + k, 8 * i + j] = 1.0
                Q_r[k, 3 * j + k, 8 * i + j] = 1.0
                P_d[k, 3 * i + k, 8 * i + 4 + j] = 1.0
                Q_d[k, 3 * j + k, 8 * i + 4 + j] = 1.0
    mask_d = np.zeros((1, 32), np.float32)
    for i in range(4):
        for j in range(4):
            mask_d[0, 8 * i + 4 + j] = 1.0
    return P_r, Q_r, P_d, Q_d, mask_d


_PR, _QR, _PD, _QD, _MASKD = _sel_mats()


def _precompute_body(h_ref, hyd_ref, chg_ref, wq_ref, bq_ref, wkm_ref,
                     wvm_ref, bv_ref, wkrT_ref, wkaT_ref, bk_ref,
                     htab_ref, ctab_ref,
                     src_ref, tk_ref, tv_ref, hc_ref):
    h = h_ref[...]
    q = jnp.dot(h, wq_ref[...], preferred_element_type=jnp.float32) + bq_ref[...]
    g = jnp.dot(q, wkrT_ref[...], preferred_element_type=jnp.float32)
    a = jnp.dot(q, wkaT_ref[...], preferred_element_type=jnp.float32)
    qb = jnp.sum(q * bk_ref[...], axis=1, keepdims=True)
    src_ref[...] = jnp.concatenate(
        [q, g, a, qb, jnp.zeros((q.shape[0], 15), jnp.float32)], axis=1)
    tk_ref[...] = jnp.dot(h, wkm_ref[...], preferred_element_type=jnp.float32)
    tv_ref[...] = jnp.dot(h, wvm_ref[...], preferred_element_type=jnp.float32) + bv_ref[...]
    ids = jax.lax.broadcasted_iota(jnp.int32, (h.shape[0], 16), 1)
    oh_h = (hyd_ref[...] == ids).astype(jnp.float32)
    oh_c = (chg_ref[...] == ids).astype(jnp.float32)
    e1 = jnp.dot(oh_h, htab_ref[...], preferred_element_type=jnp.float32)
    e2 = jnp.dot(oh_c, ctab_ref[...], preferred_element_type=jnp.float32)
    hc_ref[...] = jnp.concatenate([e1, e2], axis=1)


def _precompute(h, hyd, chg, Wq, bq, Wkm, Wvm, bv, WkrT, WkaT, bk, htab, ctab):
    nb = N // NBLK
    full = lambda shape: pl.BlockSpec(shape, lambda i: (0, 0))
    return pl.pallas_call(
        _precompute_body,
        grid=(nb,),
        in_specs=[
            pl.BlockSpec((NBLK, 128), lambda i: (i, 0)),
            pl.BlockSpec((NBLK, 1), lambda i: (i, 0)),
            pl.BlockSpec((NBLK, 1), lambda i: (i, 0)),
            full((128, 128)), full((1, 128)), full((128, 128)),
            full((128, 128)), full((1, 128)), full((128, 32)),
            full((128, 16)), full((1, 128)), full((16, 64)), full((16, 64)),
        ],
        out_specs=[
            pl.BlockSpec((NBLK, 192), lambda i: (i, 0)),
            pl.BlockSpec((NBLK, 128), lambda i: (i, 0)),
            pl.BlockSpec((NBLK, 128), lambda i: (i, 0)),
            pl.BlockSpec((NBLK, 128), lambda i: (i, 0)),
        ],
        out_shape=[
            jax.ShapeDtypeStruct((N, 192), jnp.float32),
            jax.ShapeDtypeStruct((N, 128), jnp.float32),
            jax.ShapeDtypeStruct((N, 128), jnp.float32),
            jax.ShapeDtypeStruct((N, 128), jnp.float32),
        ],
    )(h, hyd, chg, Wq, bq, Wkm, Wvm, bv, WkrT, WkaT, bk, htab, ctab)


def _coord_gather_body(cp_ref, rows_ref, cols_ref, cr_ref, cc_ref,
                       ridx, cidx, rbuf, cbuf, sem1, sem2):
    cid = lax.axis_index("c")
    sid = lax.axis_index("s")
    wid = sid * 2 + cid
    base = wid * EPW

    def chunk(j, _):
        off = base + j * CH
        pltpu.sync_copy(rows_ref.at[pl.ds(off, CH)], ridx)
        pltpu.sync_copy(cols_ref.at[pl.ds(off, CH)], cidx)
        c1 = pltpu.async_copy(cp_ref.at[ridx], rbuf, sem1)
        c2 = pltpu.async_copy(cp_ref.at[cidx], cbuf, sem2)
        c1.wait()
        c2.wait()
        pltpu.sync_copy(rbuf, cr_ref.at[pl.ds(off, CH)])
        pltpu.sync_copy(cbuf, cc_ref.at[pl.ds(off, CH)])
        return 0

    lax.fori_loop(0, NCH, chunk, 0)


def _coord_gather(cp, rows, cols):
    mesh = plsc.VectorSubcoreMesh(core_axis_name="c", subcore_axis_name="s")
    return pl.kernel(
        _coord_gather_body,
        out_type=[
            jax.ShapeDtypeStruct((E, 16), jnp.float32),
            jax.ShapeDtypeStruct((E, 16), jnp.float32),
        ],
        mesh=mesh,
        scratch_types=[
            pltpu.VMEM((CH,), jnp.int32),
            pltpu.VMEM((CH,), jnp.int32),
            pltpu.VMEM((CH, 16), jnp.float32),
            pltpu.VMEM((CH, 16), jnp.float32),
            pltpu.SemaphoreType.DMA,
            pltpu.SemaphoreType.DMA,
        ],
    )(cp, rows, cols)


def _radial_body(cr_ref, cc_ref, pr_ref, qr_ref, pd_ref, qd_ref, mask_ref,
                 raw_ref, svec_ref):
    i = pl.program_id(0)
    cr = cr_ref[...]
    cc = cc_ref[...]
    d = cr - cc
    radial = jnp.zeros((cr.shape[0], 32), jnp.float32)
    sq = jnp.zeros((cr.shape[0], 32), jnp.float32)
    for k in range(3):
        radial = radial + jnp.dot(d, pr_ref[k], preferred_element_type=jnp.float32) \
            * jnp.dot(d, qr_ref[k], preferred_element_type=jnp.float32)
        dd = jnp.dot(cr, pd_ref[k], preferred_element_type=jnp.float32) \
            - jnp.dot(cc, qd_ref[k], preferred_element_type=jnp.float32)
        sq = sq + dd * dd
    raw = radial + mask_ref[...] * jnp.sqrt(sq + 1e-12)
    raw_ref[...] = raw
    sums = jnp.sum(raw * raw, axis=0)[None, :]
    sums = jnp.pad(sums, ((0, 7), (0, 96)))

    @pl.when(i == 0)
    def _():
        svec_ref[...] = jnp.zeros_like(svec_ref)

    svec_ref[...] += sums

    @pl.when(i == pl.num_programs(0) - 1)
    def _():
        acc = svec_ref[...]
        svec_ref[...] = 1.0 / jnp.maximum(jnp.sqrt(acc), 1e-12)


def _radial(cr, cc):
    nb = E // EBLK
    return pl.pallas_call(
        _radial_body,
        grid=(nb,),
        in_specs=[
            pl.BlockSpec((EBLK, 16), lambda i: (i, 0)),
            pl.BlockSpec((EBLK, 16), lambda i: (i, 0)),
            pl.BlockSpec((3, 16, 32), lambda i: (0, 0, 0)),
            pl.BlockSpec((3, 16, 32), lambda i: (0, 0, 0)),
            pl.BlockSpec((3, 16, 32), lambda i: (0, 0, 0)),
            pl.BlockSpec((3, 16, 32), lambda i: (0, 0, 0)),
            pl.BlockSpec((1, 32), lambda i: (0, 0)),
        ],
        out_specs=[
            pl.BlockSpec((EBLK, 32), lambda i: (i, 0)),
            pl.BlockSpec((8, 128), lambda i: (0, 0)),
        ],
        out_shape=[
            jax.ShapeDtypeStruct((E, 32), jnp.float32),
            jax.ShapeDtypeStruct((8, 128), jnp.float32),
        ],
    )(cr, cc, _PR, _QR, _PD, _QD, _MASKD)


def _alpha_body(src_ref, tk_ref, raw_ref, ea_ref, s_ref, rows_ref, cols_ref,
                alpha_ref, maxp_ref,
                ridx, cidx, sbuf, tkbuf, rawbuf, eabuf, abuf, svmem, mvbuf,
                sem1, sem2):
    cid = lax.axis_index("c")
    sid = lax.axis_index("s")
    wid = sid * 2 + cid
    base = wid * EPW
    pltpu.sync_copy(s_ref, svmem)
    s0 = svmem[pl.ds(0, 16)]
    s1 = svmem[pl.ds(16, 16)]

    def chunk(j, m):
        off = base + j * CH
        pltpu.sync_copy(rows_ref.at[pl.ds(off, CH)], ridx)
        pltpu.sync_copy(cols_ref.at[pl.ds(off, CH)], cidx)
        c1 = pltpu.async_copy(src_ref.at[ridx], sbuf, sem1)
        c2 = pltpu.async_copy(tk_ref.at[cidx], tkbuf, sem2)
        pltpu.sync_copy(raw_ref.at[pl.ds(off, CH)], rawbuf)
        pltpu.sync_copy(ea_ref.at[pl.ds(off, CH)], eabuf)
        c1.wait()
        c2.wait()

        def edge(e, m):
            acc = sbuf[e, pl.ds(0, 16)] * tkbuf[e, pl.ds(0, 16)]
            for t in range(1, 8):
                acc = acc + sbuf[e, pl.ds(16 * t, 16)] * tkbuf[e, pl.ds(16 * t, 16)]
            acc = acc + (sbuf[e, pl.ds(128, 16)] * s0) * rawbuf[e, pl.ds(0, 16)]
            acc = acc + (sbuf[e, pl.ds(144, 16)] * s1) * rawbuf[e, pl.ds(16, 16)]
            acc = acc + sbuf[e, pl.ds(160, 16)] * eabuf[e, :]
            al = jnp.sum(acc, axis=0) + sbuf[e, 176]
            abuf[e] = al
            return jnp.maximum(m, al)

        m = lax.fori_loop(0, CH, edge, m)
        pltpu.sync_copy(abuf, alpha_ref.at[pl.ds(off, CH)])
        return m

    m = lax.fori_loop(0, NCH, chunk, jnp.float32(-3e38))
    mvbuf[...] = jnp.full((16,), m, jnp.float32)
    pltpu.sync_copy(mvbuf, maxp_ref.at[wid])


def _alpha(src, tk, raw, ea, s, rows, cols):
    mesh = plsc.VectorSubcoreMesh(core_axis_name="c", subcore_axis_name="s")
    return pl.kernel(
        _alpha_body,
        out_type=[
            jax.ShapeDtypeStruct((E,), jnp.float32),
            jax.ShapeDtypeStruct((NW, 16), jnp.float32),
        ],
        mesh=mesh,
        scratch_types=[
            pltpu.VMEM((CH,), jnp.int32),
            pltpu.VMEM((CH,), jnp.int32),
            pltpu.VMEM((CH, 192), jnp.float32),
            pltpu.VMEM((CH, 128), jnp.float32),
            pltpu.VMEM((CH, 32), jnp.float32),
            pltpu.VMEM((CH, 16), jnp.float32),
            pltpu.VMEM((CH,), jnp.float32),
            pltpu.VMEM((32,), jnp.float32),
            pltpu.VMEM((16,), jnp.float32),
            pltpu.SemaphoreType.DMA,
            pltpu.SemaphoreType.DMA,
        ],
    )(src, tk, raw, ea, s, rows, cols)


NROWS_SC = N // 16
CPO = 125


def _agg_body(rows_ref, cols_ref, alpha_ref, maxp_ref, tv_ref, raw_ref,
              ea_ref, zr_ref, out_ref,
              ridx, cidx, tvbuf, rawbuf, eabuf, abuf, exbuf, stage, mpbuf,
              obuf, acc_spmem, sem1):
    cid = lax.axis_index("c")
    sid = lax.axis_index("s")
    wid = sid * 2 + cid
    base = wid * EPW

    pltpu.sync_copy(maxp_ref, mpbuf)
    mv = mpbuf[0, :]
    for r in range(1, NW):
        mv = jnp.maximum(mv, mpbuf[r, :])
    C = jnp.max(mv, axis=0)

    pltpu.sync_copy(zr_ref, obuf)
    for q in range(NROWS_SC // CPO):
        pltpu.sync_copy(obuf, acc_spmem.at[pl.ds(sid * NROWS_SC + q * CPO, CPO)])
    plsc.subcore_barrier()

    def chunk(j, _):
        off = base + j * CH
        pltpu.sync_copy(rows_ref.at[pl.ds(off, CH)], ridx)
        pltpu.sync_copy(cols_ref.at[pl.ds(off, CH)], cidx)
        c1 = pltpu.async_copy(tv_ref.at[cidx], tvbuf, sem1)
        pltpu.sync_copy(raw_ref.at[pl.ds(off, CH)], rawbuf)
        pltpu.sync_copy(ea_ref.at[pl.ds(off, CH)], eabuf)
        pltpu.sync_copy(alpha_ref.at[pl.ds(off, CH)], abuf)
        for k in range(CH // 16):
            exbuf[pl.ds(16 * k, 16)] = jnp.exp(abuf[pl.ds(16 * k, 16)] - C)
        c1.wait()

        def edge(e, _):
            ex = exbuf[e]
            for t in range(8):
                stage[e, pl.ds(16 * t, 16)] = tvbuf[e, pl.ds(16 * t, 16)] * ex
            stage[e, pl.ds(128, 16)] = rawbuf[e, pl.ds(0, 16)] * ex
            stage[e, pl.ds(144, 16)] = rawbuf[e, pl.ds(16, 16)] * ex
            stage[e, pl.ds(160, 16)] = eabuf[e, :] * ex
            stage[e, pl.ds(176, 16)] = jnp.full((16,), ex, jnp.float32)
            return 0

        lax.fori_loop(0, CH, edge, 0)
        pltpu.sync_copy(stage, acc_spmem.at[ridx], add=True)
        return 0

    lax.fori_loop(0, NCH, chunk, 0)
    plsc.subcore_barrier()

    for q in range(NROWS_SC // CPO):
        r0 = sid * NROWS_SC + q * CPO
        pltpu.sync_copy(acc_spmem.at[pl.ds(r0, CPO)], obuf)
        pltpu.sync_copy(obuf, out_ref.at[cid].at[pl.ds(r0, CPO)])


def _aggregate(rows, cols, alpha, maxp, tv, raw, ea, zr):
    mesh = plsc.VectorSubcoreMesh(core_axis_name="c", subcore_axis_name="s")
    return pl.kernel(
        _agg_body,
        out_type=jax.ShapeDtypeStruct((2, N, 192), jnp.float32),
        mesh=mesh,
        scratch_types=[
            pltpu.VMEM((CH,), jnp.int32),
            pltpu.VMEM((CH,), jnp.int32),
            pltpu.VMEM((CH, 128), jnp.float32),
            pltpu.VMEM((CH, 32), jnp.float32),
            pltpu.VMEM((CH, 16), jnp.float32),
            pltpu.VMEM((CH,), jnp.float32),
            pltpu.VMEM((CH,), jnp.float32),
            pltpu.VMEM((CH, 192), jnp.float32),
            pltpu.VMEM((NW, 16), jnp.float32),
            pltpu.VMEM((CPO, 192), jnp.float32),
            pltpu.VMEM_SHARED((N, 192), jnp.float32),
            pltpu.SemaphoreType.DMA,
        ],
    )(rows, cols, alpha, maxp, tv, raw, ea, zr)


def _final_body(o0_ref, o1_ref, h_ref, hc_ref, s_ref, wvr_ref, wva_ref,
                w1_ref, b1_ref, w2_ref, b2_ref, w3_ref, b3_ref, out_ref):
    ssum = o0_ref[...] + o1_ref[...]
    s128 = ssum[:, :128]
    s32 = ssum[:, 128:160] * s_ref[...]
    s16 = ssum[:, 160:176]
    den = ssum[:, 176:177]
    agg = s128 + jnp.dot(s32, wvr_ref[...], preferred_element_type=jnp.float32) \
        + jnp.dot(s16, wva_ref[...], preferred_element_type=jnp.float32)
    agg = jnp.where(den > 0, agg / jnp.where(den > 0, den, 1.0), 0.0)
    h2 = h_ref[...] + agg
    x = jnp.concatenate([h2, hc_ref[...]], axis=1)
    x = jnp.maximum(jnp.dot(x, w1_ref[...], preferred_element_type=jnp.float32)
                    + b1_ref[...], 0.0)
    x = jnp.maximum(jnp.dot(x, w2_ref[...], preferred_element_type=jnp.float32)
                    + b2_ref[...], 0.0)
    out_ref[...] = jnp.dot(x, w3_ref[...], preferred_element_type=jnp.float32) \
        + b3_ref[...]


def _final(o0, o1, h, hc, s, Wvr, Wva, W1, b1, W2, b2, W3, b3):
    nb = N // NBLK
    full = lambda shape: pl.BlockSpec(shape, lambda i: (0, 0))
    return pl.pallas_call(
        _final_body,
        grid=(nb,),
        in_specs=[
            pl.BlockSpec((NBLK, 192), lambda i: (i, 0)),
            pl.BlockSpec((NBLK, 192), lambda i: (i, 0)),
            pl.BlockSpec((NBLK, 128), lambda i: (i, 0)),
            pl.BlockSpec((NBLK, 128), lambda i: (i, 0)),
            full((1, 32)), full((32, 128)), full((16, 128)),
            full((256, 256)), full((1, 256)), full((256, 256)),
            full((1, 256)), full((256, 128)), full((1, 128)),
        ],
        out_specs=pl.BlockSpec((NBLK, 128), lambda i: (i, 0)),
        out_shape=jax.ShapeDtypeStruct((N, 128), jnp.float32),
    )(o0, o1, h, hc, s, Wvr, Wva, W1, b1, W2, b2, W3, b3)


def kernel(h, edge_index, coord, edge_attr, hydropathy, charge,
           Wq, bq, Wkv, bkv, hyd_tab, chg_tab, W1, b1, W2, b2, W3, b3):
    rows = edge_index[0]
    cols = edge_index[1]
    cp = jnp.pad(coord.reshape(N, 12), ((0, 0), (0, 4)))

    Wk = Wkv[:, 0::2]
    Wv = Wkv[:, 1::2]
    bk = bkv[0::2][None, :]
    bv = bkv[1::2][None, :]
    Wkm = Wk[32:160]
    Wvm = Wv[32:160]
    WkrT = jnp.transpose(Wk[0:32])
    WkaT = jnp.transpose(Wk[160:176])
    Wvr = Wv[0:32]
    Wva = Wv[160:176]
    htab = jnp.pad(hyd_tab, ((0, 6), (0, 0)))
    ctab = jnp.pad(chg_tab, ((0, 6), (0, 0)))

    src, tk, tv, hc = _precompute(
        h, hydropathy[:, None], charge[:, None], Wq, bq[None, :],
        Wkm, Wvm, bv, WkrT, WkaT, bk, htab, ctab)

    cr, cc = _coord_gather(cp, rows, cols)
    raw, svec = _radial(cr, cc)
    s32 = svec[0, :32]

    alpha, maxp = _alpha(src, tk, raw, edge_attr, s32, rows, cols)

    zr = jnp.zeros((CPO, 192), jnp.float32)
    out2 = _aggregate(rows, cols, alpha, maxp, tv, raw, edge_attr, zr)

    out = _final(out2[0], out2[1], h, hc, svec[0:1, :32],
                 Wvr, Wva, W1, b1[None, :], W2, b2[None, :], W3, b3[None, :])
    return out


# speedup vs baseline: 3.1097x; 3.1097x over previous
import functools
import numpy as np
import jax
import jax.numpy as jnp
from jax import lax
from jax.experimental import pallas as pl
from jax.experimental.pallas import tpu as pltpu
from jax.experimental.pallas import tpu_sc as plsc

N = 10000
E = 320000
NW = 32
EPW = E // NW
CH = 80
NCH = EPW // CH
NBLK = 2000
EBLK = 2000


def _sel_mats():
    P_r = np.zeros((3, 16, 32), np.float32)
    Q_r = np.zeros((3, 16, 32), np.float32)
    P_d = np.zeros((3, 16, 32), np.float32)
    Q_d = np.zeros((3, 16, 32), np.float32)
    for k in range(3):
        for i in range(4):
            for j in range(4):
                P_r[k, 3 * i + k, 8 * i + j] = 1.0
                Q_r[k, 3 * j + k, 8 * i + j] = 1.0
                P_d[k, 3 * i + k, 8 * i + 4 + j] = 1.0
                Q_d[k, 3 * j + k, 8 * i + 4 + j] = 1.0
    mask_d = np.zeros((1, 32), np.float32)
    for i in range(4):
        for j in range(4):
            mask_d[0, 8 * i + 4 + j] = 1.0
    return P_r, Q_r, P_d, Q_d, mask_d


_PR, _QR, _PD, _QD, _MASKD = _sel_mats()


def _precompute_body(h_ref, hyd_ref, chg_ref, wq_ref, bq_ref, wkm_ref,
                     wvm_ref, bv_ref, wkrT_ref, wkaT_ref, bk_ref,
                     htab_ref, ctab_ref,
                     src_ref, tk_ref, tv_ref, hc_ref):
    h = h_ref[...]
    q = jnp.dot(h, wq_ref[...], preferred_element_type=jnp.float32) + bq_ref[...]
    g = jnp.dot(q, wkrT_ref[...], preferred_element_type=jnp.float32)
    a = jnp.dot(q, wkaT_ref[...], preferred_element_type=jnp.float32)
    qb = jnp.sum(q * bk_ref[...], axis=1, keepdims=True)
    src_ref[...] = jnp.concatenate(
        [q, g, a, qb, jnp.zeros((q.shape[0], 15), jnp.float32)], axis=1)
    tk_ref[...] = jnp.dot(h, wkm_ref[...], preferred_element_type=jnp.float32)
    tv_ref[...] = jnp.dot(h, wvm_ref[...], preferred_element_type=jnp.float32) + bv_ref[...]
    ids = jax.lax.broadcasted_iota(jnp.int32, (h.shape[0], 16), 1)
    oh_h = (hyd_ref[...] == ids).astype(jnp.float32)
    oh_c = (chg_ref[...] == ids).astype(jnp.float32)
    e1 = jnp.dot(oh_h, htab_ref[...], preferred_element_type=jnp.float32)
    e2 = jnp.dot(oh_c, ctab_ref[...], preferred_element_type=jnp.float32)
    hc_ref[...] = jnp.concatenate([e1, e2], axis=1)


def _precompute(h, hyd, chg, Wq, bq, Wkm, Wvm, bv, WkrT, WkaT, bk, htab, ctab):
    nb = N // NBLK
    full = lambda shape: pl.BlockSpec(shape, lambda i: (0, 0))
    return pl.pallas_call(
        _precompute_body,
        grid=(nb,),
        in_specs=[
            pl.BlockSpec((NBLK, 128), lambda i: (i, 0)),
            pl.BlockSpec((NBLK, 1), lambda i: (i, 0)),
            pl.BlockSpec((NBLK, 1), lambda i: (i, 0)),
            full((128, 128)), full((1, 128)), full((128, 128)),
            full((128, 128)), full((1, 128)), full((128, 32)),
            full((128, 16)), full((1, 128)), full((16, 64)), full((16, 64)),
        ],
        out_specs=[
            pl.BlockSpec((NBLK, 192), lambda i: (i, 0)),
            pl.BlockSpec((NBLK, 128), lambda i: (i, 0)),
            pl.BlockSpec((NBLK, 128), lambda i: (i, 0)),
            pl.BlockSpec((NBLK, 128), lambda i: (i, 0)),
        ],
        out_shape=[
            jax.ShapeDtypeStruct((N, 192), jnp.float32),
            jax.ShapeDtypeStruct((N, 128), jnp.float32),
            jax.ShapeDtypeStruct((N, 128), jnp.float32),
            jax.ShapeDtypeStruct((N, 128), jnp.float32),
        ],
    )(h, hyd, chg, Wq, bq, Wkm, Wvm, bv, WkrT, WkaT, bk, htab, ctab)


def _coord_gather_body(cp_ref, rows_ref, cols_ref, cr_ref, cc_ref,
                       ridx, cidx, rbuf, cbuf, sem1, sem2):
    cid = lax.axis_index("c")
    sid = lax.axis_index("s")
    wid = sid * 2 + cid
    base = wid * EPW

    def chunk(j, _):
        off = base + j * CH
        pltpu.sync_copy(rows_ref.at[pl.ds(off, CH)], ridx)
        pltpu.sync_copy(cols_ref.at[pl.ds(off, CH)], cidx)
        c1 = pltpu.async_copy(cp_ref.at[ridx], rbuf, sem1)
        c2 = pltpu.async_copy(cp_ref.at[cidx], cbuf, sem2)
        c1.wait()
        c2.wait()
        pltpu.sync_copy(rbuf, cr_ref.at[pl.ds(off, CH)])
        pltpu.sync_copy(cbuf, cc_ref.at[pl.ds(off, CH)])
        return 0

    lax.fori_loop(0, NCH, chunk, 0)


def _coord_gather(cp, rows, cols):
    mesh = plsc.VectorSubcoreMesh(core_axis_name="c", subcore_axis_name="s")
    return pl.kernel(
        _coord_gather_body,
        out_type=[
            jax.ShapeDtypeStruct((E, 16), jnp.float32),
            jax.ShapeDtypeStruct((E, 16), jnp.float32),
        ],
        mesh=mesh,
        compiler_params=pltpu.CompilerParams(use_tc_tiling_on_sc=False, needs_layout_passes=False),
        scratch_types=[
            pltpu.VMEM((CH,), jnp.int32),
            pltpu.VMEM((CH,), jnp.int32),
            pltpu.VMEM((CH, 16), jnp.float32),
            pltpu.VMEM((CH, 16), jnp.float32),
            pltpu.SemaphoreType.DMA,
            pltpu.SemaphoreType.DMA,
        ],
    )(cp, rows, cols)


def _radial_body(cr_ref, cc_ref, pr_ref, qr_ref, pd_ref, qd_ref, mask_ref,
                 raw_ref, svec_ref):
    i = pl.program_id(0)
    cr = cr_ref[...]
    cc = cc_ref[...]
    d = cr - cc
    radial = jnp.zeros((cr.shape[0], 32), jnp.float32)
    sq = jnp.zeros((cr.shape[0], 32), jnp.float32)
    for k in range(3):
        radial = radial + jnp.dot(d, pr_ref[k], preferred_element_type=jnp.float32) \
            * jnp.dot(d, qr_ref[k], preferred_element_type=jnp.float32)
        dd = jnp.dot(cr, pd_ref[k], preferred_element_type=jnp.float32) \
            - jnp.dot(cc, qd_ref[k], preferred_element_type=jnp.float32)
        sq = sq + dd * dd
    raw = radial + mask_ref[...] * jnp.sqrt(sq + 1e-12)
    raw_ref[...] = raw
    sums = jnp.sum(raw * raw, axis=0)[None, :]
    sums = jnp.pad(sums, ((0, 7), (0, 96)))

    @pl.when(i == 0)
    def _():
        svec_ref[...] = jnp.zeros_like(svec_ref)

    svec_ref[...] += sums

    @pl.when(i == pl.num_programs(0) - 1)
    def _():
        acc = svec_ref[...]
        svec_ref[...] = 1.0 / jnp.maximum(jnp.sqrt(acc), 1e-12)


def _radial(cr, cc):
    nb = E // EBLK
    return pl.pallas_call(
        _radial_body,
        grid=(nb,),
        in_specs=[
            pl.BlockSpec((EBLK, 16), lambda i: (i, 0)),
            pl.BlockSpec((EBLK, 16), lambda i: (i, 0)),
            pl.BlockSpec((3, 16, 32), lambda i: (0, 0, 0)),
            pl.BlockSpec((3, 16, 32), lambda i: (0, 0, 0)),
            pl.BlockSpec((3, 16, 32), lambda i: (0, 0, 0)),
            pl.BlockSpec((3, 16, 32), lambda i: (0, 0, 0)),
            pl.BlockSpec((1, 32), lambda i: (0, 0)),
        ],
        out_specs=[
            pl.BlockSpec((EBLK, 32), lambda i: (i, 0)),
            pl.BlockSpec((8, 128), lambda i: (0, 0)),
        ],
        out_shape=[
            jax.ShapeDtypeStruct((E, 32), jnp.float32),
            jax.ShapeDtypeStruct((8, 128), jnp.float32),
        ],
    )(cr, cc, _PR, _QR, _PD, _QD, _MASKD)


def _alpha_body(src_ref, tk_ref, raw_ref, ea_ref, s_ref, rows_ref, cols_ref,
                alpha_ref, maxp_ref,
                ridx, cidx, sbuf, tkbuf, rawbuf, eabuf, abuf, svmem, mvbuf,
                sem1, sem2):
    cid = lax.axis_index("c")
    sid = lax.axis_index("s")
    wid = sid * 2 + cid
    base = wid * EPW
    pltpu.sync_copy(s_ref, svmem)
    s0 = svmem[pl.ds(0, 16)]
    s1 = svmem[pl.ds(16, 16)]

    def chunk(j, m):
        off = base + j * CH
        pltpu.sync_copy(rows_ref.at[pl.ds(off, CH)], ridx)
        pltpu.sync_copy(cols_ref.at[pl.ds(off, CH)], cidx)
        c1 = pltpu.async_copy(src_ref.at[ridx], sbuf, sem1)
        c2 = pltpu.async_copy(tk_ref.at[cidx], tkbuf, sem2)
        pltpu.sync_copy(raw_ref.at[pl.ds(off, CH)], rawbuf)
        pltpu.sync_copy(ea_ref.at[pl.ds(off, CH)], eabuf)
        c1.wait()
        c2.wait()

        lanes = lax.iota(jnp.int32, 16)

        def group(g, m):
            avec = jnp.zeros((16,), jnp.float32)
            for i in range(16):
                e = g * 16 + i
                acc = sbuf[e, pl.ds(0, 16)] * tkbuf[e, pl.ds(0, 16)]
                for t in range(1, 8):
                    acc = acc + sbuf[e, pl.ds(16 * t, 16)] * tkbuf[e, pl.ds(16 * t, 16)]
                acc = acc + (sbuf[e, pl.ds(128, 16)] * s0) * rawbuf[e, pl.ds(0, 16)]
                acc = acc + (sbuf[e, pl.ds(144, 16)] * s1) * rawbuf[e, pl.ds(16, 16)]
                acc = acc + sbuf[e, pl.ds(160, 16)] * eabuf[e, :]
                al = jnp.sum(acc, axis=0) + sbuf[e, pl.ds(176, 16)][0]
                avec = jnp.where(lanes == i, al, avec)
                m = jnp.maximum(m, al)
            abuf[pl.ds(g * 16, 16)] = avec
            return m

        m = lax.fori_loop(0, CH // 16, group, m)
        pltpu.sync_copy(abuf, alpha_ref.at[pl.ds(off, CH)])
        return m

    m = lax.fori_loop(0, NCH, chunk, jnp.float32(-3e38))
    mvbuf[...] = jnp.full((16,), m, jnp.float32)
    pltpu.sync_copy(mvbuf, maxp_ref.at[wid])


def _alpha(src, tk, raw, ea, s, rows, cols):
    mesh = plsc.VectorSubcoreMesh(core_axis_name="c", subcore_axis_name="s")
    return pl.kernel(
        _alpha_body,
        out_type=[
            jax.ShapeDtypeStruct((E,), jnp.float32),
            jax.ShapeDtypeStruct((NW, 16), jnp.float32),
        ],
        mesh=mesh,
        compiler_params=pltpu.CompilerParams(use_tc_tiling_on_sc=False, needs_layout_passes=False),
        scratch_types=[
            pltpu.VMEM((CH,), jnp.int32),
            pltpu.VMEM((CH,), jnp.int32),
            pltpu.VMEM((CH, 192), jnp.float32),
            pltpu.VMEM((CH, 128), jnp.float32),
            pltpu.VMEM((CH, 32), jnp.float32),
            pltpu.VMEM((CH, 16), jnp.float32),
            pltpu.VMEM((CH,), jnp.float32),
            pltpu.VMEM((32,), jnp.float32),
            pltpu.VMEM((16,), jnp.float32),
            pltpu.SemaphoreType.DMA,
            pltpu.SemaphoreType.DMA,
        ],
    )(src, tk, raw, ea, s, rows, cols)


NHALF = 5000
NTAB = 5120
EPT = E // 16
NCHB = EPT // CH
ZROWS = NTAB // 16


def _agg_body(rows_ref, cols_ref, alpha_ref, maxp_ref, tv_ref, raw_ref,
              ea_ref, zr_ref, out_ref,
              ridx, cidx, lidx, tvbuf, rawbuf, eabuf, abuf, stage, mpbuf,
              obuf, acc_spmem, sem1):
    cid = lax.axis_index("c")
    sid = lax.axis_index("s")
    base = sid * EPT
    nbase = cid * NHALF

    pltpu.sync_copy(maxp_ref, mpbuf)
    mv = mpbuf[0, :]
    for r in range(1, NW):
        mv = jnp.maximum(mv, mpbuf[r, :])
    C = jnp.max(mv, axis=0)

    pltpu.sync_copy(zr_ref, obuf)
    for q in range(ZROWS // 160):
        pltpu.sync_copy(obuf, acc_spmem.at[pl.ds(sid * ZROWS + q * 160, 160)])
    plsc.subcore_barrier()

    def chunk(j, _):
        off = base + j * CH
        pltpu.sync_copy(rows_ref.at[pl.ds(off, CH)], ridx)
        pltpu.sync_copy(cols_ref.at[pl.ds(off, CH)], cidx)
        c1 = pltpu.async_copy(tv_ref.at[cidx], tvbuf, sem1)
        pltpu.sync_copy(raw_ref.at[pl.ds(off, CH)], rawbuf)
        pltpu.sync_copy(ea_ref.at[pl.ds(off, CH)], eabuf)
        pltpu.sync_copy(alpha_ref.at[pl.ds(off, CH)], abuf)
        for k in range(CH // 16):
            lr = ridx[pl.ds(16 * k, 16)] - nbase
            ok = (lr >= 0) & (lr < NHALF)
            lidx[pl.ds(16 * k, 16)] = jnp.where(ok, lr, NHALF)
        c1.wait()

        def group(g, _):
            exg = jnp.exp(abuf[pl.ds(g * 16, 16)] - C)
            for i in range(16):
                e = g * 16 + i
                ex = exg[i]
                for t in range(8):
                    stage[e, pl.ds(16 * t, 16)] = tvbuf[e, pl.ds(16 * t, 16)] * ex
                stage[e, pl.ds(128, 16)] = rawbuf[e, pl.ds(0, 16)] * ex
                stage[e, pl.ds(144, 16)] = rawbuf[e, pl.ds(16, 16)] * ex
                stage[e, pl.ds(160, 16)] = eabuf[e, :] * ex
                stage[e, pl.ds(176, 16)] = jnp.zeros((16,), jnp.float32) + ex
            return 0

        lax.fori_loop(0, CH // 16, group, 0)
        pltpu.sync_copy(stage, acc_spmem.at[lidx], add=True)
        return 0

    lax.fori_loop(0, NCHB, chunk, 0)
    plsc.subcore_barrier()

    for q in range(ZROWS // 160):
        r0 = sid * ZROWS + q * 160
        pltpu.sync_copy(acc_spmem.at[pl.ds(r0, 160)], obuf)
        pltpu.sync_copy(obuf, out_ref.at[cid].at[pl.ds(r0, 160)])


def _aggregate(rows, cols, alpha, maxp, tv, raw, ea, zr):
    mesh = plsc.VectorSubcoreMesh(core_axis_name="c", subcore_axis_name="s")
    return pl.kernel(
        _agg_body,
        out_type=jax.ShapeDtypeStruct((2, NTAB, 192), jnp.float32),
        mesh=mesh,
        compiler_params=pltpu.CompilerParams(use_tc_tiling_on_sc=False, needs_layout_passes=False),
        scratch_types=[
            pltpu.VMEM((CH,), jnp.int32),
            pltpu.VMEM((CH,), jnp.int32),
            pltpu.VMEM((CH,), jnp.int32),
            pltpu.VMEM((CH, 128), jnp.float32),
            pltpu.VMEM((CH, 32), jnp.float32),
            pltpu.VMEM((CH, 16), jnp.float32),
            pltpu.VMEM((CH,), jnp.float32),
            pltpu.VMEM((CH, 192), jnp.float32),
            pltpu.VMEM((NW, 16), jnp.float32),
            pltpu.VMEM((160, 192), jnp.float32),
            pltpu.VMEM_SHARED((NTAB, 192), jnp.float32),
            pltpu.SemaphoreType.DMA,
        ],
    )(rows, cols, alpha, maxp, tv, raw, ea, zr)


def _final_body(o0_ref, h_ref, hc_ref, s_ref, wvr_ref, wva_ref,
                w1_ref, b1_ref, w2_ref, b2_ref, w3_ref, b3_ref, out_ref):
    ssum = o0_ref[...]
    s128 = ssum[:, :128]
    s32 = ssum[:, 128:160] * s_ref[...]
    s16 = ssum[:, 160:176]
    den = ssum[:, 176:177]
    agg = s128 + jnp.dot(s32, wvr_ref[...], preferred_element_type=jnp.float32) \
        + jnp.dot(s16, wva_ref[...], preferred_element_type=jnp.float32)
    agg = jnp.where(den > 0, agg / jnp.where(den > 0, den, 1.0), 0.0)
    h2 = h_ref[...] + agg
    x = jnp.concatenate([h2, hc_ref[...]], axis=1)
    x = jnp.maximum(jnp.dot(x, w1_ref[...], preferred_element_type=jnp.float32)
                    + b1_ref[...], 0.0)
    x = jnp.maximum(jnp.dot(x, w2_ref[...], preferred_element_type=jnp.float32)
                    + b2_ref[...], 0.0)
    out_ref[...] = jnp.dot(x, w3_ref[...], preferred_element_type=jnp.float32) \
        + b3_ref[...]


def _final(o0, h, hc, s, Wvr, Wva, W1, b1, W2, b2, W3, b3):
    nb = N // NBLK
    full = lambda shape: pl.BlockSpec(shape, lambda i: (0, 0))
    return pl.pallas_call(
        _final_body,
        grid=(nb,),
        in_specs=[
            pl.BlockSpec((NBLK, 192), lambda i: (i, 0)),
            pl.BlockSpec((NBLK, 128), lambda i: (i, 0)),
            pl.BlockSpec((NBLK, 128), lambda i: (i, 0)),
            full((1, 32)), full((32, 128)), full((16, 128)),
            full((256, 256)), full((1, 256)), full((256, 256)),
            full((1, 256)), full((256, 128)), full((1, 128)),
        ],
        out_specs=pl.BlockSpec((NBLK, 128), lambda i: (i, 0)),
        out_shape=jax.ShapeDtypeStruct((N, 128), jnp.float32),
    )(o0, h, hc, s, Wvr, Wva, W1, b1, W2, b2, W3, b3)


def kernel(h, edge_index, coord, edge_attr, hydropathy, charge,
           Wq, bq, Wkv, bkv, hyd_tab, chg_tab, W1, b1, W2, b2, W3, b3):
    rows = edge_index[0]
    cols = edge_index[1]
    cp = jnp.pad(coord.reshape(N, 12), ((0, 0), (0, 4)))

    Wk = Wkv[:, 0::2]
    Wv = Wkv[:, 1::2]
    bk = bkv[0::2][None, :]
    bv = bkv[1::2][None, :]
    Wkm = Wk[32:160]
    Wvm = Wv[32:160]
    WkrT = jnp.transpose(Wk[0:32])
    WkaT = jnp.transpose(Wk[160:176])
    Wvr = Wv[0:32]
    Wva = Wv[160:176]
    htab = jnp.pad(hyd_tab, ((0, 6), (0, 0)))
    ctab = jnp.pad(chg_tab, ((0, 6), (0, 0)))

    src, tk, tv, hc = _precompute(
        h, hydropathy[:, None], charge[:, None], Wq, bq[None, :],
        Wkm, Wvm, bv, WkrT, WkaT, bk, htab, ctab)

    cr, cc = _coord_gather(cp, rows, cols)
    raw, svec = _radial(cr, cc)
    s32 = svec[0, :32]

    alpha, maxp = _alpha(src, tk, raw, edge_attr, s32, rows, cols)

    zr = jnp.zeros((160, 192), jnp.float32)
    out2 = _aggregate(rows, cols, alpha, maxp, tv, raw, edge_attr, zr)
    ofull = jnp.concatenate([out2[0, :NHALF], out2[1, :NHALF]], axis=0)

    out = _final(ofull, h, hc, svec[0:1, :32],
                 Wvr, Wva, W1, b1[None, :], W2, b2[None, :], W3, b3[None, :])
    return out
